# baseline (speedup 1.0000x reference)
import functools

import jax
import jax.numpy as jnp
from jax import lax
from jax.experimental import pallas as pl
from jax.experimental.pallas import tpu as pltpu
from jax.experimental.pallas import tpu_sc as plsc

NC = 2
NS = 16
CHUNK = 88
NBUF = 4
NOUT = 2
UNROLL = 4


def _mesh():
    return plsc.VectorSubcoreMesh(
        core_axis_name="c", subcore_axis_name="s", num_cores=NC, num_subcores=NS
    )


_SC_PARAMS = pltpu.CompilerParams(needs_layout_passes=False)
_SC_PARAMS_NT = pltpu.CompilerParams(needs_layout_passes=False,
                                     use_tc_tiling_on_sc=False)


@functools.lru_cache(maxsize=None)
def _make_deg(e_pad: int, n_pad: int):
    per_tile_chunks = e_pad // (NC * NS * CHUNK)
    rows_per_tile = n_pad // NS

    @functools.partial(
        pl.kernel,
        out_type=jax.ShapeDtypeStruct((NC, n_pad), jnp.float32),
        mesh=_mesh(),
        scratch_types=[
            pltpu.VMEM_SHARED((NS, n_pad), jnp.float32),
            pltpu.VMEM((n_pad,), jnp.float32),
            pltpu.VMEM((3, CHUNK), jnp.int32),
            pltpu.VMEM((3, CHUNK), jnp.int32),
            pltpu.VMEM((rows_per_tile,), jnp.float32),
            pltpu.VMEM((rows_per_tile,), jnp.float32),
            pltpu.SemaphoreType.DMA,
            pltpu.SemaphoreType.DMA,
        ],
        compiler_params=_SC_PARAMS,
    )
    def deg_k(ep_hbm, pdeg_hbm, part_sh, deg_v, ev0, ev1, acc_v, tmp_v,
              si0, si1):
        cid = lax.axis_index("c")
        sid = lax.axis_index("s")
        row0 = sid * rows_per_tile
        base = (sid * NC + cid) * per_tile_chunks

        @plsc.parallel_loop(0, n_pad // 16)
        def _zero(i):
            deg_v[pl.ds(i * 16, 16)] = jnp.zeros((16,), jnp.float32)

        ev = (ev0, ev1)
        si = (si0, si1)
        pltpu.async_copy(ep_hbm.at[base], ev0, si0)
        pltpu.async_copy(ep_hbm.at[base + 1], ev1, si1)

        tail = CHUNK % 16

        def accum(b):
            for j in range(CHUNK // 16):
                sl = pl.ds(j * 16, 16)
                dj = ev[b][1, sl]
                wj = plsc.bitcast(ev[b][2, sl], jnp.float32)
                plsc.addupdate_scatter(deg_v, [dj], wj)
            if tail:
                sl = pl.ds(CHUNK - 16, 16)
                dj = ev[b][1, sl]
                wj = plsc.bitcast(ev[b][2, sl], jnp.float32)
                mask = lax.iota(jnp.int32, 16) >= (16 - tail)
                plsc.addupdate_scatter(deg_v, [dj], wj, mask=mask)

        def step(k, carry):
            for b in range(2):
                kk = k * 2 + b
                pltpu.make_async_copy(ep_hbm.at[base], ev[b], si[b]).wait()
                accum(b)

                @pl.when(kk + 2 < per_tile_chunks)
                def _():
                    pltpu.async_copy(ep_hbm.at[base + kk + 2], ev[b], si[b])
            return carry

        lax.fori_loop(0, per_tile_chunks // 2, step, 0)
        if per_tile_chunks % 2:
            b = (per_tile_chunks - 1) % 2
            pltpu.make_async_copy(ep_hbm.at[base], ev[b], si[b]).wait()
            accum(b)

        pltpu.sync_copy(deg_v, part_sh.at[sid])
        plsc.subcore_barrier()
        land = (tmp_v, deg_v.at[pl.ds(0, rows_per_tile)])
        sems = (si0, si1)
        sl_rows = pl.ds(row0, rows_per_tile)

        @plsc.parallel_loop(0, rows_per_tile // 16)
        def _zacc(j):
            acc_v[pl.ds(j * 16, 16)] = jnp.zeros((16,), jnp.float32)

        pltpu.async_copy(part_sh.at[0, sl_rows], land[0], sems[0])
        pltpu.async_copy(part_sh.at[1, sl_rows], land[1], sems[1])
        for t in range(NS):
            b = t % 2
            pltpu.make_async_copy(part_sh.at[t, sl_rows], land[b],
                                  sems[b]).wait()
            for j in range(rows_per_tile // 16):
                sl = pl.ds(j * 16, 16)
                acc_v[sl] += land[b][sl]
            if t + 2 < NS:
                pltpu.async_copy(part_sh.at[t + 2, sl_rows], land[b], sems[b])
        pltpu.sync_copy(acc_v, pdeg_hbm.at[cid, pl.ds(row0, rows_per_tile)])

    return deg_k


@functools.lru_cache(maxsize=None)
def _make_mp(e_pad: int, n_rows: int, ch: int, split: float = 0.5):
    n2 = e_pad // (NS * CHUNK)
    n_a = int(round(n2 * split / UNROLL)) * UNROLL
    n_a = min(max(n_a, UNROLL), n2 - UNROLL)
    assert n2 % UNROLL == 0
    rows_per_tile = n_rows // NS
    chp = ch // 2
    hmask = jnp.int32(-65536)

    @functools.partial(
        pl.kernel,
        out_type=jax.ShapeDtypeStruct((NC, n_rows, ch), jnp.float32),
        mesh=_mesh(),
        scratch_types=[
            pltpu.VMEM_SHARED((n_rows, ch), jnp.float32),
            [pltpu.VMEM((3, CHUNK), jnp.int32) for _ in range(NBUF)],
            [pltpu.VMEM((CHUNK, chp), jnp.int32) for _ in range(NBUF)],
            [pltpu.VMEM((CHUNK, ch), jnp.float32) for _ in range(NOUT)],
            [pltpu.VMEM((CHUNK,), jnp.int32) for _ in range(NOUT)],
            [pltpu.SemaphoreType.DMA for _ in range(NBUF)],
            [pltpu.SemaphoreType.DMA for _ in range(NBUF)],
            [pltpu.SemaphoreType.DMA for _ in range(NOUT)],
        ],
        compiler_params=_SC_PARAMS_NT,
    )
    def mp_k(ep_hbm, gp_hbm, out_hbm, acc_sh, ev, pk, fo, dv, si, sg, ss):
        cid = lax.axis_index("c")
        sid = lax.axis_index("s")
        row0 = sid * rows_per_tile
        n = jnp.where(cid == 0, n_a, n2 - n_a)
        base = sid * n2 + cid * n_a

        nq = ch // 16

        @plsc.parallel_loop(0, CHUNK * nq)
        def _zero(t):
            fo[0][t // nq, pl.ds((t % nq) * 16, 16)] = (
                jnp.zeros((16,), jnp.float32))

        left = rows_per_tile
        off = 0
        while left > 0:
            blk = min(CHUNK, left)
            pltpu.sync_copy(fo[0].at[pl.ds(0, blk)],
                            acc_sh.at[pl.ds(row0 + off, blk)])
            off += blk
            left -= blk
        plsc.subcore_barrier()

        def compute(X, Z):
            for j in range(CHUNK // 16):
                sl = pl.ds(j * 16, 16)
                dv[Z][sl] = ev[X][1, sl]
            if CHUNK % 16:
                sl = pl.ds(CHUNK - 16, 16)
                dv[Z][sl] = ev[X][1, sl]

            @plsc.parallel_loop(0, CHUNK)
            def _scale(e):
                bc = plsc.bitcast(
                    plsc.load_gather(
                        ev[X].at[2], [jnp.full((16,), 0, jnp.int32) + e]),
                    jnp.float32)
                for q in range(ch // 32):
                    sl = pl.ds(q * 16, 16)
                    v = pk[X][e, sl]
                    lo = plsc.bitcast(v << 16, jnp.float32)
                    hi = plsc.bitcast(v & hmask, jnp.float32)
                    fo[Z][e, pl.ds(q * 16, 16)] = lo * bc
                    fo[Z][e, pl.ds(ch // 2 + q * 16, 16)] = hi * bc

        for b in range(NBUF):
            pltpu.async_copy(ep_hbm.at[base + b], ev[b], si[b])
        for b in range(NBUF - 1):
            pltpu.make_async_copy(ep_hbm.at[base], ev[b], si[b]).wait()
            pltpu.async_copy(gp_hbm.at[ev[b].at[0]], pk[b], sg[b])

        def step(kk, carry):
            for u in range(UNROLL):
                k = kk * UNROLL + u
                X = u % NBUF
                XL = (u + NBUF - 1) % NBUF
                Z = u % NOUT

                @pl.when(k + NBUF - 1 < n)
                def _():
                    pltpu.make_async_copy(
                        ep_hbm.at[base], ev[XL], si[XL]).wait()
                    pltpu.async_copy(gp_hbm.at[ev[XL].at[0]], pk[XL], sg[XL])

                pltpu.make_async_copy(
                    gp_hbm.at[ev[X].at[0]], pk[X], sg[X]).wait()

                @pl.when(k >= NOUT)
                def _():
                    pltpu.make_async_copy(
                        fo[Z], acc_sh.at[dv[Z]], ss[Z]).wait()

                compute(X, Z)
                pltpu.async_copy(fo[Z], acc_sh.at[dv[Z]], ss[Z], add=True)

                @pl.when(k + NBUF < n)
                def _():
                    pltpu.async_copy(ep_hbm.at[base + k + NBUF], ev[X], si[X])
            return carry

        lax.fori_loop(0, n // UNROLL, step, 0)
        for z in range(NOUT):
            pltpu.make_async_copy(fo[z], acc_sh.at[dv[z]], ss[z]).wait()

        plsc.subcore_barrier()
        pltpu.sync_copy(
            acc_sh.at[pl.ds(row0, rows_per_tile)],
            out_hbm.at[cid, pl.ds(row0, rows_per_tile)],
        )

    return mp_k


@functools.lru_cache(maxsize=None)
def _make_matmul(n: int, k: int, m: int):
    br = 1000 if n % 1000 == 0 else 8
    grid = n // br

    def body(x_ref, w_ref, dinv_ref, o_ref, op_ref):
        g = jnp.dot(x_ref[...], w_ref[...],
                    preferred_element_type=jnp.float32) * dinv_ref[...]
        o_ref[...] = g

        def rne(bits):
            return bits + jnp.int32(0x7FFF) + (
                lax.shift_right_logical(bits, 16) & 1)

        bl = rne(lax.bitcast_convert_type(g[:, :m // 2], jnp.int32))
        bh = rne(lax.bitcast_convert_type(g[:, m // 2:], jnp.int32))
        op_ref[...] = (bh & jnp.int32(-65536)) | lax.shift_right_logical(bl, 16)

    return pl.pallas_call(
        body,
        grid=(grid,),
        in_specs=[
            pl.BlockSpec((br, k), lambda i: (i, 0)),
            pl.BlockSpec((k, m), lambda i: (0, 0)),
            pl.BlockSpec((br, 1), lambda i: (i, 0)),
        ],
        out_specs=[
            pl.BlockSpec((br, m), lambda i: (i, 0)),
            pl.BlockSpec((br, m // 2), lambda i: (i, 0)),
        ],
        out_shape=[
            jax.ShapeDtypeStruct((n, m), jnp.float32),
            jax.ShapeDtypeStruct((n, m // 2), jnp.int32),
        ],
    )


@functools.lru_cache(maxsize=None)
def _make_batchnorm(n: int, ch: int):
    br = 1000 if n % 1000 == 0 else 8
    grid = n // br

    def body(acc_ref, g_ref, dinv_ref, gamma_ref, beta_ref, o_ref, scr):
        p = pl.program_id(0)
        i = pl.program_id(1)
        a = (acc_ref[0] + acc_ref[1] + g_ref[...]) * dinv_ref[...]

        @pl.when(jnp.logical_and(p == 0, i == 0))
        def _():
            scr[...] = jnp.zeros_like(scr)

        @pl.when(p == 0)
        def _():
            scr[0:1, :] += jnp.sum(a, axis=0, keepdims=True)
            scr[1:2, :] += jnp.sum(a * a, axis=0, keepdims=True)

            @pl.when(i == grid - 1)
            def _():
                mean = scr[0:1, :] * (1.0 / n)
                var = scr[1:2, :] * (1.0 / n) - mean * mean
                scale = gamma_ref[...] * jax.lax.rsqrt(var + 1e-5)
                scr[0:1, :] = scale
                scr[1:2, :] = beta_ref[...] - mean * scale

        @pl.when(p == 1)
        def _():
            o_ref[...] = jnp.maximum(a * scr[0:1, :] + scr[1:2, :], 0.0)

    return pl.pallas_call(
        body,
        grid=(2, grid),
        in_specs=[
            pl.BlockSpec((2, br, ch), lambda p, i: (0, i, 0)),
            pl.BlockSpec((br, ch), lambda p, i: (i, 0)),
            pl.BlockSpec((br, 1), lambda p, i: (i, 0)),
            pl.BlockSpec((1, ch), lambda p, i: (0, 0)),
            pl.BlockSpec((1, ch), lambda p, i: (0, 0)),
        ],
        out_specs=pl.BlockSpec((br, ch), lambda p, i: (i, 0)),
        out_shape=jax.ShapeDtypeStruct((n, ch), jnp.float32),
        scratch_shapes=[pltpu.VMEM((2, ch), jnp.float32)],
    )


def kernel(x, edge_index, edge_weight, W, b, gamma, beta):
    n, k = x.shape
    m = W.shape[1]
    e = edge_weight.shape[0]

    src = edge_index[0].astype(jnp.int32)
    dst = edge_index[1].astype(jnp.int32)
    w = edge_weight.astype(jnp.float32)
    step = NC * NS * CHUNK * NBUF
    e_pad = -(-e // step) * step
    pad = e_pad - e
    src_p = jnp.pad(src, (0, pad)).reshape(-1, CHUNK)
    dst_p = jnp.pad(dst, (0, pad)).reshape(-1, CHUNK)
    w_p = lax.bitcast_convert_type(
        jnp.pad(w, (0, pad)), jnp.int32).reshape(-1, CHUNK)
    ep = jnp.stack([src_p, dst_p, w_p], axis=1)
    n_pad = -(-n // 256) * 256
    n_rows = -(-n // 128) * 128

    pdeg = _make_deg(e_pad, n_pad)(ep)
    deg = pdeg[0, :n] + pdeg[1, :n] + 1.0
    dinv = jax.lax.rsqrt(jnp.maximum(deg, 1e-30))
    dinv2 = dinv.reshape(n, 1)

    g, gp = _make_matmul(n, k, m)(x, W, dinv2)
    acc = _make_mp(e_pad, n_rows, m, 2.0 / 3.0)(ep, gp)

    out = _make_batchnorm(n, m)(
        acc, g, dinv2, gamma.reshape(1, m), beta.reshape(1, m))
    return out


# speedup vs baseline: 1.1629x; 1.1629x over previous
import functools

import jax
import jax.numpy as jnp
from jax import lax
from jax.experimental import pallas as pl
from jax.experimental.pallas import tpu as pltpu
from jax.experimental.pallas import tpu_sc as plsc

NC = 2
NS = 16
CHUNK = 96
NBUF = 3
NOUT = 2
UNROLL = 6


def _mesh():
    return plsc.VectorSubcoreMesh(
        core_axis_name="c", subcore_axis_name="s", num_cores=NC, num_subcores=NS
    )


_SC_PARAMS = pltpu.CompilerParams(needs_layout_passes=False)
_SC_PARAMS_NT = pltpu.CompilerParams(needs_layout_passes=False,
                                     use_tc_tiling_on_sc=False)


@functools.lru_cache(maxsize=None)
def _make_deg(e_pad: int, n_pad: int):
    per_tile_chunks = e_pad // (NC * NS * CHUNK)
    rows_per_tile = n_pad // NS

    @functools.partial(
        pl.kernel,
        out_type=jax.ShapeDtypeStruct((NC, n_pad), jnp.float32),
        mesh=_mesh(),
        scratch_types=[
            pltpu.VMEM_SHARED((NS, n_pad), jnp.float32),
            pltpu.VMEM((n_pad,), jnp.float32),
            pltpu.VMEM((3, CHUNK), jnp.int32),
            pltpu.VMEM((3, CHUNK), jnp.int32),
            pltpu.VMEM((rows_per_tile,), jnp.float32),
            pltpu.VMEM((rows_per_tile,), jnp.float32),
            pltpu.SemaphoreType.DMA,
            pltpu.SemaphoreType.DMA,
        ],
        compiler_params=_SC_PARAMS,
    )
    def deg_k(ep_hbm, pdeg_hbm, part_sh, deg_v, ev0, ev1, acc_v, tmp_v,
              si0, si1):
        cid = lax.axis_index("c")
        sid = lax.axis_index("s")
        row0 = sid * rows_per_tile
        base = (sid * NC + cid) * per_tile_chunks

        @plsc.parallel_loop(0, n_pad // 16)
        def _zero(i):
            deg_v[pl.ds(i * 16, 16)] = jnp.zeros((16,), jnp.float32)

        ev = (ev0, ev1)
        si = (si0, si1)
        pltpu.async_copy(ep_hbm.at[base], ev0, si0)
        pltpu.async_copy(ep_hbm.at[base + 1], ev1, si1)

        tail = CHUNK % 16

        def accum(b):
            for j in range(CHUNK // 16):
                sl = pl.ds(j * 16, 16)
                dj = ev[b][1, sl]
                wj = plsc.bitcast(ev[b][2, sl], jnp.float32)
                plsc.addupdate_scatter(deg_v, [dj], wj)
            if tail:
                sl = pl.ds(CHUNK - 16, 16)
                dj = ev[b][1, sl]
                wj = plsc.bitcast(ev[b][2, sl], jnp.float32)
                mask = lax.iota(jnp.int32, 16) >= (16 - tail)
                plsc.addupdate_scatter(deg_v, [dj], wj, mask=mask)

        def step(k, carry):
            for b in range(2):
                kk = k * 2 + b
                pltpu.make_async_copy(ep_hbm.at[base], ev[b], si[b]).wait()
                accum(b)

                @pl.when(kk + 2 < per_tile_chunks)
                def _():
                    pltpu.async_copy(ep_hbm.at[base + kk + 2], ev[b], si[b])
            return carry

        lax.fori_loop(0, per_tile_chunks // 2, step, 0)
        if per_tile_chunks % 2:
            b = (per_tile_chunks - 1) % 2
            pltpu.make_async_copy(ep_hbm.at[base], ev[b], si[b]).wait()
            accum(b)

        pltpu.sync_copy(deg_v, part_sh.at[sid])
        plsc.subcore_barrier()
        land = (tmp_v, deg_v.at[pl.ds(0, rows_per_tile)])
        sems = (si0, si1)
        sl_rows = pl.ds(row0, rows_per_tile)

        @plsc.parallel_loop(0, rows_per_tile // 16)
        def _zacc(j):
            acc_v[pl.ds(j * 16, 16)] = jnp.zeros((16,), jnp.float32)

        pltpu.async_copy(part_sh.at[0, sl_rows], land[0], sems[0])
        pltpu.async_copy(part_sh.at[1, sl_rows], land[1], sems[1])
        for t in range(NS):
            b = t % 2
            pltpu.make_async_copy(part_sh.at[t, sl_rows], land[b],
                                  sems[b]).wait()
            for j in range(rows_per_tile // 16):
                sl = pl.ds(j * 16, 16)
                acc_v[sl] += land[b][sl]
            if t + 2 < NS:
                pltpu.async_copy(part_sh.at[t + 2, sl_rows], land[b], sems[b])
        pltpu.sync_copy(acc_v, pdeg_hbm.at[cid, pl.ds(row0, rows_per_tile)])

    return deg_k


@functools.lru_cache(maxsize=None)
def _make_mp(e_pad: int, n_rows: int, ch: int, split: float = 0.5):
    n2 = e_pad // (NS * CHUNK)
    n_a = int(round(n2 * split / UNROLL)) * UNROLL
    n_a = min(max(n_a, UNROLL), n2 - UNROLL)
    assert n2 % UNROLL == 0
    rows_per_tile = n_rows // NS
    chp = ch // 2
    hmask = jnp.int32(-65536)

    @functools.partial(
        pl.kernel,
        out_type=jax.ShapeDtypeStruct((NC, n_rows, ch), jnp.float32),
        mesh=_mesh(),
        scratch_types=[
            pltpu.VMEM_SHARED((n_rows, ch), jnp.float32),
            [pltpu.VMEM((3, CHUNK), jnp.int32) for _ in range(NBUF)],
            [pltpu.VMEM((CHUNK, chp), jnp.int32) for _ in range(NBUF)],
            [pltpu.VMEM((CHUNK, ch), jnp.float32) for _ in range(NOUT)],
            [pltpu.VMEM((CHUNK,), jnp.int32) for _ in range(NOUT)],
            [pltpu.SemaphoreType.DMA for _ in range(NBUF)],
            [pltpu.SemaphoreType.DMA for _ in range(NBUF)],
            [pltpu.SemaphoreType.DMA for _ in range(NOUT)],
        ],
        compiler_params=_SC_PARAMS_NT,
    )
    def mp_k(ep_hbm, gp_hbm, out_hbm, acc_sh, ev, pk, fo, dv, si, sg, ss):
        cid = lax.axis_index("c")
        sid = lax.axis_index("s")
        row0 = sid * rows_per_tile
        n = jnp.where(cid == 0, n_a, n2 - n_a)
        base = sid * n2 + cid * n_a

        nq = ch // 16

        @plsc.parallel_loop(0, CHUNK * nq)
        def _zero(t):
            fo[0][t // nq, pl.ds((t % nq) * 16, 16)] = (
                jnp.zeros((16,), jnp.float32))

        left = rows_per_tile
        off = 0
        while left > 0:
            blk = min(CHUNK, left)
            pltpu.sync_copy(fo[0].at[pl.ds(0, blk)],
                            acc_sh.at[pl.ds(row0 + off, blk)])
            off += blk
            left -= blk
        plsc.subcore_barrier()

        def compute(X, Z):
            for j in range(CHUNK // 16):
                sl = pl.ds(j * 16, 16)
                dv[Z][sl] = ev[X][1, sl]
            if CHUNK % 16:
                sl = pl.ds(CHUNK - 16, 16)
                dv[Z][sl] = ev[X][1, sl]

            @plsc.parallel_loop(0, CHUNK)
            def _scale(e):
                bc = plsc.bitcast(
                    plsc.load_gather(
                        ev[X].at[2], [jnp.full((16,), 0, jnp.int32) + e]),
                    jnp.float32)
                for q in range(ch // 32):
                    sl = pl.ds(q * 16, 16)
                    v = pk[X][e, sl]
                    lo = plsc.bitcast(v << 16, jnp.float32)
                    hi = plsc.bitcast(v & hmask, jnp.float32)
                    fo[Z][e, pl.ds(q * 16, 16)] = lo * bc
                    fo[Z][e, pl.ds(ch // 2 + q * 16, 16)] = hi * bc

        for b in range(NBUF):
            pltpu.async_copy(ep_hbm.at[base + b], ev[b], si[b])
        for b in range(NBUF - 1):
            pltpu.make_async_copy(ep_hbm.at[base], ev[b], si[b]).wait()
            pltpu.async_copy(gp_hbm.at[ev[b].at[0]], pk[b], sg[b])

        def step(kk, carry):
            for u in range(UNROLL):
                k = kk * UNROLL + u
                X = u % NBUF
                XL = (u + NBUF - 1) % NBUF
                Z = u % NOUT

                @pl.when(k + NBUF - 1 < n)
                def _():
                    pltpu.make_async_copy(
                        ep_hbm.at[base], ev[XL], si[XL]).wait()
                    pltpu.async_copy(gp_hbm.at[ev[XL].at[0]], pk[XL], sg[XL])

                pltpu.make_async_copy(
                    gp_hbm.at[ev[X].at[0]], pk[X], sg[X]).wait()

                @pl.when(k >= NOUT)
                def _():
                    pltpu.make_async_copy(
                        fo[Z], acc_sh.at[dv[Z]], ss[Z]).wait()

                compute(X, Z)
                pltpu.async_copy(fo[Z], acc_sh.at[dv[Z]], ss[Z], add=True)

                @pl.when(k + NBUF < n)
                def _():
                    pltpu.async_copy(ep_hbm.at[base + k + NBUF], ev[X], si[X])
            return carry

        lax.fori_loop(0, n // UNROLL, step, 0)
        for z in range(NOUT):
            pltpu.make_async_copy(fo[z], acc_sh.at[dv[z]], ss[z]).wait()

        plsc.subcore_barrier()
        pltpu.sync_copy(
            acc_sh.at[pl.ds(row0, rows_per_tile)],
            out_hbm.at[cid, pl.ds(row0, rows_per_tile)],
        )

    return mp_k


@functools.lru_cache(maxsize=None)
def _make_matmul(n: int, k: int, m: int):
    br = 1000 if n % 1000 == 0 else 8
    grid = n // br

    def body(x_ref, w_ref, dinv_ref, o_ref, op_ref):
        g = jnp.dot(x_ref[...], w_ref[...],
                    preferred_element_type=jnp.float32) * dinv_ref[...]
        o_ref[...] = g

        def rne(bits):
            return bits + jnp.int32(0x7FFF) + (
                lax.shift_right_logical(bits, 16) & 1)

        bl = rne(lax.bitcast_convert_type(g[:, :m // 2], jnp.int32))
        bh = rne(lax.bitcast_convert_type(g[:, m // 2:], jnp.int32))
        op_ref[...] = (bh & jnp.int32(-65536)) | lax.shift_right_logical(bl, 16)

    return pl.pallas_call(
        body,
        grid=(grid,),
        in_specs=[
            pl.BlockSpec((br, k), lambda i: (i, 0)),
            pl.BlockSpec((k, m), lambda i: (0, 0)),
            pl.BlockSpec((br, 1), lambda i: (i, 0)),
        ],
        out_specs=[
            pl.BlockSpec((br, m), lambda i: (i, 0)),
            pl.BlockSpec((br, m // 2), lambda i: (i, 0)),
        ],
        out_shape=[
            jax.ShapeDtypeStruct((n, m), jnp.float32),
            jax.ShapeDtypeStruct((n, m // 2), jnp.int32),
        ],
    )


@functools.lru_cache(maxsize=None)
def _make_batchnorm(n: int, ch: int):
    br = 1000 if n % 1000 == 0 else 8
    grid = n // br

    def body(acc_ref, g_ref, dinv_ref, gamma_ref, beta_ref, o_ref, scr):
        p = pl.program_id(0)
        i = pl.program_id(1)
        a = (acc_ref[0] + acc_ref[1] + g_ref[...]) * dinv_ref[...]

        @pl.when(jnp.logical_and(p == 0, i == 0))
        def _():
            scr[...] = jnp.zeros_like(scr)

        @pl.when(p == 0)
        def _():
            scr[0:1, :] += jnp.sum(a, axis=0, keepdims=True)
            scr[1:2, :] += jnp.sum(a * a, axis=0, keepdims=True)

            @pl.when(i == grid - 1)
            def _():
                mean = scr[0:1, :] * (1.0 / n)
                var = scr[1:2, :] * (1.0 / n) - mean * mean
                scale = gamma_ref[...] * jax.lax.rsqrt(var + 1e-5)
                scr[0:1, :] = scale
                scr[1:2, :] = beta_ref[...] - mean * scale

        @pl.when(p == 1)
        def _():
            o_ref[...] = jnp.maximum(a * scr[0:1, :] + scr[1:2, :], 0.0)

    return pl.pallas_call(
        body,
        grid=(2, grid),
        in_specs=[
            pl.BlockSpec((2, br, ch), lambda p, i: (0, i, 0)),
            pl.BlockSpec((br, ch), lambda p, i: (i, 0)),
            pl.BlockSpec((br, 1), lambda p, i: (i, 0)),
            pl.BlockSpec((1, ch), lambda p, i: (0, 0)),
            pl.BlockSpec((1, ch), lambda p, i: (0, 0)),
        ],
        out_specs=pl.BlockSpec((br, ch), lambda p, i: (i, 0)),
        out_shape=jax.ShapeDtypeStruct((n, ch), jnp.float32),
        scratch_shapes=[pltpu.VMEM((2, ch), jnp.float32)],
    )


def kernel(x, edge_index, edge_weight, W, b, gamma, beta):
    n, k = x.shape
    m = W.shape[1]
    e = edge_weight.shape[0]

    src = edge_index[0].astype(jnp.int32)
    dst = edge_index[1].astype(jnp.int32)
    w = edge_weight.astype(jnp.float32)
    step = NC * NS * CHUNK * NBUF
    e_pad = -(-e // step) * step
    pad = e_pad - e
    src_p = jnp.pad(src, (0, pad)).reshape(-1, CHUNK)
    dst_p = jnp.pad(dst, (0, pad)).reshape(-1, CHUNK)
    w_p = lax.bitcast_convert_type(
        jnp.pad(w, (0, pad)), jnp.int32).reshape(-1, CHUNK)
    ep = jnp.stack([src_p, dst_p, w_p], axis=1)
    n_pad = -(-n // 256) * 256
    n_rows = -(-n // 128) * 128

    pdeg = _make_deg(e_pad, n_pad)(ep)
    deg = pdeg[0, :n] + pdeg[1, :n] + 1.0
    dinv = jax.lax.rsqrt(jnp.maximum(deg, 1e-30))
    dinv2 = dinv.reshape(n, 1)

    g, gp = _make_matmul(n, k, m)(x, W, dinv2)
    acc = _make_mp(e_pad, n_rows, m, 2.0 / 3.0)(ep, gp)

    out = _make_batchnorm(n, m)(
        acc, g, dinv2, gamma.reshape(1, m), beta.reshape(1, m))
    return out


# speedup vs baseline: 1.1698x; 1.0060x over previous
import functools

import jax
import jax.numpy as jnp
from jax import lax
from jax.experimental import pallas as pl
from jax.experimental.pallas import tpu as pltpu
from jax.experimental.pallas import tpu_sc as plsc

NC = 2
NS = 16
CHUNK = 96
NBUF = 3
NOUT = 2
UNROLL = 6


def _mesh():
    return plsc.VectorSubcoreMesh(
        core_axis_name="c", subcore_axis_name="s", num_cores=NC, num_subcores=NS
    )


_SC_PARAMS = pltpu.CompilerParams(needs_layout_passes=False)
_SC_PARAMS_NT = pltpu.CompilerParams(needs_layout_passes=False,
                                     use_tc_tiling_on_sc=False)


@functools.lru_cache(maxsize=None)
def _make_deg(e_pad: int, n_pad: int):
    per_tile_chunks = e_pad // (NC * NS * CHUNK)
    rows_per_tile = n_pad // NS

    @functools.partial(
        pl.kernel,
        out_type=jax.ShapeDtypeStruct((NC, n_pad), jnp.float32),
        mesh=_mesh(),
        scratch_types=[
            pltpu.VMEM_SHARED((NS, n_pad), jnp.float32),
            pltpu.VMEM((n_pad,), jnp.float32),
            pltpu.VMEM((3, CHUNK), jnp.int32),
            pltpu.VMEM((3, CHUNK), jnp.int32),
            pltpu.VMEM((rows_per_tile,), jnp.float32),
            pltpu.VMEM((rows_per_tile,), jnp.float32),
            pltpu.SemaphoreType.DMA,
            pltpu.SemaphoreType.DMA,
        ],
        compiler_params=_SC_PARAMS,
    )
    def deg_k(ep_hbm, pdeg_hbm, part_sh, deg_v, ev0, ev1, acc_v, tmp_v,
              si0, si1):
        cid = lax.axis_index("c")
        sid = lax.axis_index("s")
        row0 = sid * rows_per_tile
        base = (sid * NC + cid) * per_tile_chunks

        @plsc.parallel_loop(0, n_pad // 16)
        def _zero(i):
            deg_v[pl.ds(i * 16, 16)] = jnp.zeros((16,), jnp.float32)

        ev = (ev0, ev1)
        si = (si0, si1)
        pltpu.async_copy(ep_hbm.at[base], ev0, si0)
        pltpu.async_copy(ep_hbm.at[base + 1], ev1, si1)

        tail = CHUNK % 16

        def accum(b):
            for j in range(CHUNK // 16):
                sl = pl.ds(j * 16, 16)
                dj = ev[b][1, sl]
                wj = plsc.bitcast(ev[b][2, sl], jnp.float32)
                plsc.addupdate_scatter(deg_v, [dj], wj)
            if tail:
                sl = pl.ds(CHUNK - 16, 16)
                dj = ev[b][1, sl]
                wj = plsc.bitcast(ev[b][2, sl], jnp.float32)
                mask = lax.iota(jnp.int32, 16) >= (16 - tail)
                plsc.addupdate_scatter(deg_v, [dj], wj, mask=mask)

        def step(k, carry):
            for b in range(2):
                kk = k * 2 + b
                pltpu.make_async_copy(ep_hbm.at[base], ev[b], si[b]).wait()
                accum(b)

                @pl.when(kk + 2 < per_tile_chunks)
                def _():
                    pltpu.async_copy(ep_hbm.at[base + kk + 2], ev[b], si[b])
            return carry

        lax.fori_loop(0, per_tile_chunks // 2, step, 0)
        if per_tile_chunks % 2:
            b = (per_tile_chunks - 1) % 2
            pltpu.make_async_copy(ep_hbm.at[base], ev[b], si[b]).wait()
            accum(b)

        pltpu.sync_copy(deg_v, part_sh.at[sid])
        plsc.subcore_barrier()
        land = (tmp_v, deg_v.at[pl.ds(0, rows_per_tile)])
        sems = (si0, si1)
        sl_rows = pl.ds(row0, rows_per_tile)

        @plsc.parallel_loop(0, rows_per_tile // 16)
        def _zacc(j):
            acc_v[pl.ds(j * 16, 16)] = jnp.zeros((16,), jnp.float32)

        pltpu.async_copy(part_sh.at[0, sl_rows], land[0], sems[0])
        pltpu.async_copy(part_sh.at[1, sl_rows], land[1], sems[1])
        for t in range(NS):
            b = t % 2
            pltpu.make_async_copy(part_sh.at[t, sl_rows], land[b],
                                  sems[b]).wait()
            for j in range(rows_per_tile // 16):
                sl = pl.ds(j * 16, 16)
                acc_v[sl] += land[b][sl]
            if t + 2 < NS:
                pltpu.async_copy(part_sh.at[t + 2, sl_rows], land[b], sems[b])
        pltpu.sync_copy(acc_v, pdeg_hbm.at[cid, pl.ds(row0, rows_per_tile)])

    return deg_k


@functools.lru_cache(maxsize=None)
def _make_mp(e_pad: int, n_rows: int, ch: int, split: float = 0.5):
    n2 = e_pad // (NS * CHUNK)
    n_a = int(round(n2 * split / UNROLL)) * UNROLL
    n_a = min(max(n_a, UNROLL), n2 - UNROLL)
    assert n2 % UNROLL == 0
    rows_per_tile = n_rows // NS
    chp = ch // 2
    hmask = jnp.int32(-65536)

    @functools.partial(
        pl.kernel,
        out_type=jax.ShapeDtypeStruct((NC, n_rows, ch), jnp.float32),
        mesh=_mesh(),
        scratch_types=[
            pltpu.VMEM_SHARED((n_rows, ch), jnp.float32),
            [pltpu.VMEM((3, CHUNK), jnp.int32) for _ in range(NBUF)],
            [pltpu.VMEM((CHUNK, chp), jnp.int32) for _ in range(NBUF)],
            [pltpu.VMEM((CHUNK, ch), jnp.float32) for _ in range(NOUT)],
            [pltpu.VMEM((CHUNK,), jnp.int32) for _ in range(NOUT)],
            [pltpu.SemaphoreType.DMA for _ in range(NBUF)],
            [pltpu.SemaphoreType.DMA for _ in range(NBUF)],
            [pltpu.SemaphoreType.DMA for _ in range(NOUT)],
        ],
        compiler_params=_SC_PARAMS_NT,
    )
    def mp_k(ep_hbm, gp_hbm, out_hbm, acc_sh, ev, pk, fo, dv, si, sg, ss):
        cid = lax.axis_index("c")
        sid = lax.axis_index("s")
        row0 = sid * rows_per_tile
        n = jnp.where(cid == 0, n_a, n2 - n_a)
        base = sid * n2 + cid * n_a

        nq = ch // 16

        @plsc.parallel_loop(0, CHUNK * nq)
        def _zero(t):
            fo[0][t // nq, pl.ds((t % nq) * 16, 16)] = (
                jnp.zeros((16,), jnp.float32))

        left = rows_per_tile
        off = 0
        while left > 0:
            blk = min(CHUNK, left)
            pltpu.sync_copy(fo[0].at[pl.ds(0, blk)],
                            acc_sh.at[pl.ds(row0 + off, blk)])
            off += blk
            left -= blk
        plsc.subcore_barrier()

        def compute(X, Z):
            for j in range(CHUNK // 16):
                sl = pl.ds(j * 16, 16)
                dv[Z][sl] = ev[X][1, sl]
            if CHUNK % 16:
                sl = pl.ds(CHUNK - 16, 16)
                dv[Z][sl] = ev[X][1, sl]

            @plsc.parallel_loop(0, CHUNK)
            def _scale(e):
                bc = plsc.bitcast(
                    plsc.load_gather(
                        ev[X].at[2], [jnp.full((16,), 0, jnp.int32) + e]),
                    jnp.float32)
                for q in range(ch // 32):
                    sl = pl.ds(q * 16, 16)
                    v = pk[X][e, sl]
                    lo = plsc.bitcast(v << 16, jnp.float32)
                    hi = plsc.bitcast(v & hmask, jnp.float32)
                    fo[Z][e, pl.ds(q * 16, 16)] = lo * bc
                    fo[Z][e, pl.ds(ch // 2 + q * 16, 16)] = hi * bc

        for b in range(NBUF):
            pltpu.async_copy(ep_hbm.at[base + b], ev[b], si[b])
        for b in range(NBUF - 1):
            pltpu.make_async_copy(ep_hbm.at[base], ev[b], si[b]).wait()
            pltpu.async_copy(gp_hbm.at[ev[b].at[0]], pk[b], sg[b])

        def step(kk, carry):
            for u in range(UNROLL):
                k = kk * UNROLL + u
                X = u % NBUF
                XL = (u + NBUF - 1) % NBUF
                Z = u % NOUT

                @pl.when(k + NBUF - 1 < n)
                def _():
                    pltpu.make_async_copy(
                        ep_hbm.at[base], ev[XL], si[XL]).wait()
                    pltpu.async_copy(gp_hbm.at[ev[XL].at[0]], pk[XL], sg[XL])

                pltpu.make_async_copy(
                    gp_hbm.at[ev[X].at[0]], pk[X], sg[X]).wait()

                @pl.when(k >= NOUT)
                def _():
                    pltpu.make_async_copy(
                        fo[Z], acc_sh.at[dv[Z]], ss[Z]).wait()

                compute(X, Z)
                pltpu.async_copy(fo[Z], acc_sh.at[dv[Z]], ss[Z], add=True)

                @pl.when(k + NBUF < n)
                def _():
                    pltpu.async_copy(ep_hbm.at[base + k + NBUF], ev[X], si[X])
            return carry

        lax.fori_loop(0, n // UNROLL, step, 0)
        for z in range(NOUT):
            pltpu.make_async_copy(fo[z], acc_sh.at[dv[z]], ss[z]).wait()

        plsc.subcore_barrier()
        pltpu.sync_copy(
            acc_sh.at[pl.ds(row0, rows_per_tile)],
            out_hbm.at[cid, pl.ds(row0, rows_per_tile)],
        )

    return mp_k


@functools.lru_cache(maxsize=None)
def _make_matmul(n: int, k: int, m: int):
    br = 1000 if n % 1000 == 0 else 8
    grid = n // br

    def body(x_ref, w_ref, dinv_ref, op_ref):
        g = jnp.dot(x_ref[...], w_ref[...],
                    preferred_element_type=jnp.float32) * dinv_ref[...]

        def rne(bits):
            return bits + jnp.int32(0x7FFF) + (
                lax.shift_right_logical(bits, 16) & 1)

        bl = rne(lax.bitcast_convert_type(g[:, :m // 2], jnp.int32))
        bh = rne(lax.bitcast_convert_type(g[:, m // 2:], jnp.int32))
        op_ref[...] = (bh & jnp.int32(-65536)) | lax.shift_right_logical(bl, 16)

    return pl.pallas_call(
        body,
        grid=(grid,),
        in_specs=[
            pl.BlockSpec((br, k), lambda i: (i, 0)),
            pl.BlockSpec((k, m), lambda i: (0, 0)),
            pl.BlockSpec((br, 1), lambda i: (i, 0)),
        ],
        out_specs=pl.BlockSpec((br, m // 2), lambda i: (i, 0)),
        out_shape=jax.ShapeDtypeStruct((n, m // 2), jnp.int32),
    )


@functools.lru_cache(maxsize=None)
def _make_batchnorm(n: int, ch: int):
    br = 1000 if n % 1000 == 0 else 8
    grid = n // br

    def body(acc_ref, gp_ref, dinv_ref, gamma_ref, beta_ref, o_ref, scr):
        p = pl.program_id(0)
        i = pl.program_id(1)
        gp = gp_ref[...]
        g = jnp.concatenate(
            [lax.bitcast_convert_type(gp << 16, jnp.float32),
             lax.bitcast_convert_type(gp & jnp.int32(-65536), jnp.float32)],
            axis=1)
        a = (acc_ref[0] + acc_ref[1] + g) * dinv_ref[...]

        @pl.when(jnp.logical_and(p == 0, i == 0))
        def _():
            scr[...] = jnp.zeros_like(scr)

        @pl.when(p == 0)
        def _():
            scr[0:1, :] += jnp.sum(a, axis=0, keepdims=True)
            scr[1:2, :] += jnp.sum(a * a, axis=0, keepdims=True)

            @pl.when(i == grid - 1)
            def _():
                mean = scr[0:1, :] * (1.0 / n)
                var = scr[1:2, :] * (1.0 / n) - mean * mean
                scale = gamma_ref[...] * jax.lax.rsqrt(var + 1e-5)
                scr[0:1, :] = scale
                scr[1:2, :] = beta_ref[...] - mean * scale

        @pl.when(p == 1)
        def _():
            o_ref[...] = jnp.maximum(a * scr[0:1, :] + scr[1:2, :], 0.0)

    return pl.pallas_call(
        body,
        grid=(2, grid),
        in_specs=[
            pl.BlockSpec((2, br, ch), lambda p, i: (0, i, 0)),
            pl.BlockSpec((br, ch // 2), lambda p, i: (i, 0)),
            pl.BlockSpec((br, 1), lambda p, i: (i, 0)),
            pl.BlockSpec((1, ch), lambda p, i: (0, 0)),
            pl.BlockSpec((1, ch), lambda p, i: (0, 0)),
        ],
        out_specs=pl.BlockSpec((br, ch), lambda p, i: (i, 0)),
        out_shape=jax.ShapeDtypeStruct((n, ch), jnp.float32),
        scratch_shapes=[pltpu.VMEM((2, ch), jnp.float32)],
    )


def kernel(x, edge_index, edge_weight, W, b, gamma, beta):
    n, k = x.shape
    m = W.shape[1]
    e = edge_weight.shape[0]

    src = edge_index[0].astype(jnp.int32)
    dst = edge_index[1].astype(jnp.int32)
    w = edge_weight.astype(jnp.float32)
    step = NC * NS * CHUNK * NBUF
    e_pad = -(-e // step) * step
    pad = e_pad - e
    src_p = jnp.pad(src, (0, pad)).reshape(-1, CHUNK)
    dst_p = jnp.pad(dst, (0, pad)).reshape(-1, CHUNK)
    w_p = lax.bitcast_convert_type(
        jnp.pad(w, (0, pad)), jnp.int32).reshape(-1, CHUNK)
    ep = jnp.stack([src_p, dst_p, w_p], axis=1)
    n_pad = -(-n // 256) * 256
    n_rows = -(-n // 128) * 128

    pdeg = _make_deg(e_pad, n_pad)(ep)
    deg = pdeg[0, :n] + pdeg[1, :n] + 1.0
    dinv = jax.lax.rsqrt(jnp.maximum(deg, 1e-30))
    dinv2 = dinv.reshape(n, 1)

    gp = _make_matmul(n, k, m)(x, W, dinv2)
    acc = _make_mp(e_pad, n_rows, m, 2.0 / 3.0)(ep, gp)

    out = _make_batchnorm(n, m)(
        acc, gp, dinv2, gamma.reshape(1, m), beta.reshape(1, m))
    return out
